# baseline (speedup 1.0000x reference)
import jax
import jax.numpy as jnp
from jax import lax
from jax.experimental import pallas as pl
from jax.experimental.pallas import tpu as pltpu
from jax.experimental.pallas import tpu_sc as plsc

N = 50000
D = 128
E = 400000
H = 32

CH = 112
NB = 224
PE = CH * NB
E_PAD = 16 * PE
TOT_NB = 16 * NB
K = 4
NGRP = NB // K
NSUP = NGRP // 2
ACC_N = 50176
RPT = ACC_N // 16
NZ = RPT // CH

BLK = 2000
GRID = N // BLK


def _l2n(v):
    nrm = jnp.sqrt(jnp.sum(v * v, axis=1, keepdims=True))
    return v / jnp.maximum(nrm, 1e-12)


def _fill_rows0(rows, val16):
    def fb(i, c):
        rows[0, 0, i, pl.ds(0, 16)] = val16
        rows[0, 0, i, pl.ds(16, 16)] = val16
        return c
    lax.fori_loop(0, CH, fb, 0)


def _zero_acc(rows, acc, row0):
    def zc(i, c):
        pltpu.sync_copy(rows.at[0, 0], acc.at[pl.ds(row0 + i * CH, CH)])
        return c
    lax.fori_loop(0, NZ, zc, 0)


def _seg_pipeline(tbl, cidx, ridx, core, nb0, cbk, rbk, rows, acc,
                  gsems, ssems, isems):
    def _gathers(bank):
        return [pltpu.async_copy(tbl.at[cbk.at[bank, b]],
                                 rows.at[bank, b], gsems[bank])
                for b in range(K)]

    def _drain_gathers(bank):
        for b in range(K):
            pltpu.make_async_copy(tbl.at[cbk.at[bank, b]],
                                  rows.at[bank, b], gsems[bank]).wait()

    def _scatters(bank):
        return [pltpu.async_copy(rows.at[bank, b],
                                 acc.at[rbk.at[bank, b]], ssems[bank],
                                 add=True)
                for b in range(K)]

    for bank in range(2):
        off = nb0 + bank * K
        pltpu.sync_copy(cidx.at[core, pl.ds(off, K)], cbk.at[bank])
        pltpu.sync_copy(ridx.at[core, pl.ds(off, K)], rbk.at[bank])
        _gathers(bank)

    def sup(s, c):
        sds = []
        cst = []
        for bank in range(2):
            _drain_gathers(bank)
            sds.append(_scatters(bank))
            nxt = nb0 + (2 * s + 2 + bank) * K
            cst.append(pltpu.async_copy(cidx.at[core, pl.ds(nxt, K)],
                                        cbk.at[bank], isems[bank]))
        rst = []
        for bank in range(2):
            for d in sds[bank]:
                d.wait()
            nxt = nb0 + (2 * s + 2 + bank) * K
            rst.append(pltpu.async_copy(ridx.at[core, pl.ds(nxt, K)],
                                        rbk.at[bank], isems[bank]))
            cst[bank].wait()
            _gathers(bank)
        for d in rst:
            d.wait()
        return c
    lax.fori_loop(0, NSUP - 1, sup, 0)

    for bank in range(2):
        _drain_gathers(bank)
        for d in _scatters(bank):
            d.wait()


def _cnt_pipeline(src, ridx, core, nb0, rbk, acc, ssems, isems):
    def _scat(bank):
        return [pltpu.async_copy(src, acc.at[rbk.at[bank, b]], ssems[bank],
                                 add=True)
                for b in range(K)]

    for bank in range(2):
        pltpu.sync_copy(ridx.at[core, pl.ds(nb0 + bank * K, K)],
                        rbk.at[bank])

    def sup(s, c):
        sds = [_scat(0), _scat(1)]
        rst = []
        for bank in range(2):
            for d in sds[bank]:
                d.wait()
            nxt = nb0 + (2 * s + 2 + bank) * K
            rst.append(pltpu.async_copy(ridx.at[core, pl.ds(nxt, K)],
                                        rbk.at[bank], isems[bank]))
        for d in rst:
            d.wait()
        return c
    lax.fori_loop(0, NSUP - 1, sup, 0)

    for bank in range(2):
        for d in _scat(bank):
            d.wait()


def _base_counts_body(table2, cidx, ridx, out, cbk, rbk, rows, acc,
                      gsem0, gsem1, ssem0, ssem1, isem0, isem1):
    core = lax.axis_index("c")
    sid = lax.axis_index("s")
    gsems = (gsem0, gsem1)
    ssems = (ssem0, ssem1)
    isems = (isem0, isem1)
    row0 = sid * RPT
    nb0 = sid * NB
    zero16 = jnp.zeros((16,), jnp.float32)
    one16 = jnp.ones((16,), jnp.float32)

    _fill_rows0(rows, zero16)
    _zero_acc(rows, acc, row0)
    plsc.subcore_barrier()
    _seg_pipeline(table2.at[core], cidx, ridx, core, nb0, cbk, rbk, rows,
                  acc, gsems, ssems, isems)
    plsc.subcore_barrier()
    pltpu.sync_copy(acc.at[pl.ds(row0, RPT)],
                    out.at[0, core, pl.ds(row0, RPT)])

    _fill_rows0(rows, zero16)
    _zero_acc(rows, acc, row0)
    _fill_rows0(rows, one16)
    plsc.subcore_barrier()
    _cnt_pipeline(rows.at[0, 0], ridx, core, nb0, rbk, acc, ssems, isems)
    plsc.subcore_barrier()
    pltpu.sync_copy(acc.at[pl.ds(row0, RPT)],
                    out.at[1, core, pl.ds(row0, RPT)])


def _deep_seg_body(table2, cidx, ridx, out, cbk, rbk, rows, acc,
                   gsem0, gsem1, ssem0, ssem1, isem0, isem1):
    core = lax.axis_index("c")
    sid = lax.axis_index("s")
    gsems = (gsem0, gsem1)
    ssems = (ssem0, ssem1)
    isems = (isem0, isem1)
    row0 = sid * RPT
    nb0 = sid * NB
    zero16 = jnp.zeros((16,), jnp.float32)

    for p in range(2):
        _fill_rows0(rows, zero16)
        _zero_acc(rows, acc, row0)
        plsc.subcore_barrier()
        _seg_pipeline(table2.at[p], cidx, ridx, core, nb0, cbk, rbk, rows,
                      acc, gsems, ssems, isems)
        plsc.subcore_barrier()
        pltpu.sync_copy(acc.at[pl.ds(row0, RPT)],
                        out.at[p, core, pl.ds(row0, RPT)])


def _make_sc_kernels():
    mesh = plsc.VectorSubcoreMesh(core_axis_name="c", subcore_axis_name="s")
    params = pltpu.CompilerParams(use_tc_tiling_on_sc=False)
    scratch = [
        pltpu.VMEM((2, K, CH), jnp.int32),
        pltpu.VMEM((2, K, CH), jnp.int32),
        pltpu.VMEM((2, K, CH, H), jnp.float32),
        pltpu.VMEM_SHARED((ACC_N, H), jnp.float32),
        pltpu.SemaphoreType.DMA,
        pltpu.SemaphoreType.DMA,
        pltpu.SemaphoreType.DMA,
        pltpu.SemaphoreType.DMA,
        pltpu.SemaphoreType.DMA,
        pltpu.SemaphoreType.DMA,
    ]
    base_counts = pl.kernel(
        _base_counts_body,
        out_type=jax.ShapeDtypeStruct((2, 2, ACC_N, H), jnp.float32),
        mesh=mesh,
        compiler_params=params,
        scratch_types=scratch,
    )
    deep_seg = pl.kernel(
        _deep_seg_body,
        out_type=jax.ShapeDtypeStruct((2, 2, ACC_N, H), jnp.float32),
        mesh=mesh,
        compiler_params=params,
        scratch_types=scratch,
    )
    return base_counts, deep_seg


def _proj_body(x_ref, w_ref, b_ref, zt_ref, xs_ref):
    y = jnp.dot(x_ref[...], w_ref[...], preferred_element_type=jnp.float32)
    y = y + b_ref[...]
    zt_ref[0] = y[:, 0:H]
    zt_ref[1] = y[:, H:2 * H]
    xs_ref[...] = y[:, 2 * H:]


def _base_body(bc_ref, xs_ref, h_ref):
    dp = jnp.maximum(bc_ref[1, 0, :, 0:1], 1.0)
    dn = jnp.maximum(bc_ref[1, 1, :, 0:1], 1.0)
    h_ref[0] = _l2n(bc_ref[0, 0] / dp + xs_ref[:, 0:H])
    h_ref[1] = _l2n(bc_ref[0, 1] / dn + xs_ref[:, H:])


def _deep_math(s4_ref, bc_ref, h_ref, wp_ref, bp_ref, wn_ref, bn_ref):
    cp0 = bc_ref[1, 0, :, 0:1]
    cn0 = bc_ref[1, 1, :, 0:1]
    dp = jnp.maximum(cp0, 1.0)
    dn = jnp.maximum(cn0, 1.0)
    da = jnp.maximum(cp0 + cn0, 1.0)
    sa0, sa1 = s4_ref[0, 0], s4_ref[0, 1]
    sb0, sb1 = s4_ref[1, 0], s4_ref[1, 1]
    mp_p = sa0 / dp
    mn_p = sa1 / dn
    mp_n = sb0 / dp
    mn_n = sb1 / dn
    ma_p = (sa0 + sa1) / da
    ma_n = (sb0 + sb1) / da
    hp, hn = h_ref[0], h_ref[1]
    catp = jnp.concatenate([mp_p, mn_n, mp_n, mn_p, hp, hn, ma_p], axis=1)
    catn = jnp.concatenate([mn_p, mp_n, mn_n, mp_p, hn, hp, ma_n], axis=1)
    hp2 = _l2n(jnp.dot(catp, wp_ref[...], preferred_element_type=jnp.float32)
               + bp_ref[...])
    hn2 = _l2n(jnp.dot(catn, wn_ref[...], preferred_element_type=jnp.float32)
               + bn_ref[...])
    return hp2, hn2


def _deep_mid_body(s4_ref, bc_ref, h_ref, wp_ref, bp_ref, wn_ref, bn_ref,
                   o_ref):
    hp2, hn2 = _deep_math(s4_ref, bc_ref, h_ref, wp_ref, bp_ref, wn_ref,
                          bn_ref)
    o_ref[0] = hp2
    o_ref[1] = hn2


def _deep_final_body(s4_ref, bc_ref, h_ref, wp_ref, bp_ref, wn_ref, bn_ref,
                     o_ref):
    hp2, hn2 = _deep_math(s4_ref, bc_ref, h_ref, wp_ref, bp_ref, wn_ref,
                          bn_ref)
    o_ref[...] = jnp.concatenate([hp2, hn2], axis=1)


def _spec_rows(shape):
    lead = shape[:-2]
    blk = lead + (BLK, shape[-1])

    def imap(i):
        return (0,) * len(lead) + (i, 0)
    return pl.BlockSpec(blk, imap)


def _spec_full(shape):
    return pl.BlockSpec(shape, lambda i: (0,) * len(shape))


def _proj_call(x, w, b):
    return pl.pallas_call(
        _proj_body,
        grid=(GRID,),
        in_specs=[_spec_rows(x.shape), _spec_full(w.shape), _spec_full(b.shape)],
        out_specs=[_spec_rows((2, N, H)), _spec_rows((N, 2 * H))],
        out_shape=[jax.ShapeDtypeStruct((2, N, H), jnp.float32),
                   jax.ShapeDtypeStruct((N, 2 * H), jnp.float32)],
    )(x, w, b)


def _base_call(bc, xs):
    return pl.pallas_call(
        _base_body,
        grid=(GRID,),
        in_specs=[_spec_rows(bc.shape), _spec_rows(xs.shape)],
        out_specs=_spec_rows((2, N, H)),
        out_shape=jax.ShapeDtypeStruct((2, N, H), jnp.float32),
    )(bc, xs)


def _deep_call(body, out_shape, s4, bc, h, wp, bp, wn, bn):
    return pl.pallas_call(
        body,
        grid=(GRID,),
        in_specs=[_spec_rows(s4.shape), _spec_rows(bc.shape),
                  _spec_rows(h.shape),
                  _spec_full(wp.shape), _spec_full(bp.shape),
                  _spec_full(wn.shape), _spec_full(bn.shape)],
        out_specs=_spec_rows(out_shape),
        out_shape=jax.ShapeDtypeStruct(out_shape, jnp.float32),
    )(s4, bc, h, wp, bp, wn, bn)


def kernel(x, edge_index_pos, edge_index_neg, W_pos_base, b_pos_base,
           W_neg_base, b_neg_base, W_pos_deep_0, b_pos_deep_0, W_pos_deep_1,
           b_pos_deep_1, W_neg_deep_0, b_neg_deep_0, W_neg_deep_1,
           b_neg_deep_1):
    base_counts, deep_seg = _make_sc_kernels()

    eip = edge_index_pos.astype(jnp.int32)
    ein = edge_index_neg.astype(jnp.int32)
    rp, cp = eip[0], eip[1]
    rn, cn = ein[0], ein[1]

    pad = E_PAD - E
    cpad = jnp.zeros((pad,), jnp.int32)
    rpad = jnp.full((pad,), N, jnp.int32)
    cidx = jnp.stack([jnp.concatenate([cp, cpad]),
                      jnp.concatenate([cn, cpad])]).reshape(2, TOT_NB, CH)
    ridx = jnp.stack([jnp.concatenate([rp, rpad]),
                      jnp.concatenate([rn, rpad])]).reshape(2, TOT_NB, CH)

    wcat = jnp.concatenate([W_pos_base[:D], W_neg_base[:D],
                            W_pos_base[D:], W_neg_base[D:]], axis=1)
    bcat = jnp.concatenate([jnp.zeros((2 * H,), jnp.float32),
                            b_pos_base, b_neg_base]).reshape(1, 4 * H)
    zt, xs = _proj_call(x, wcat, bcat)

    bc = base_counts(zt, cidx, ridx)
    h = _base_call(bc, xs)

    wp = [W_pos_deep_0, W_pos_deep_1]
    bp = [b_pos_deep_0.reshape(1, H), b_pos_deep_1.reshape(1, H)]
    wn = [W_neg_deep_0, W_neg_deep_1]
    bn = [b_neg_deep_0.reshape(1, H), b_neg_deep_1.reshape(1, H)]

    for i in range(2):
        s4 = deep_seg(h, cidx, ridx)
        if i == 0:
            h = _deep_call(_deep_mid_body, (2, N, H), s4, bc, h,
                           wp[i], bp[i], wn[i], bn[i])
        else:
            out = _deep_call(_deep_final_body, (N, 2 * H), s4, bc, h,
                             wp[i], bp[i], wn[i], bn[i])
    return out


# speedup vs baseline: 1.0381x; 1.0381x over previous
import jax
import jax.numpy as jnp
from jax import lax
from jax.experimental import pallas as pl
from jax.experimental.pallas import tpu as pltpu
from jax.experimental.pallas import tpu_sc as plsc

N = 50000
D = 128
E = 400000
H = 32

CH = 112
NB = 224
PE = CH * NB
E_PAD = 16 * PE
TOT_NB = 16 * NB
K = 4
NGRP = NB // K
NSUP = NGRP // 2
ACC_N = 50176
RPT = ACC_N // 16
NZ = RPT // CH

BLK = 2000
GRID = N // BLK


def _l2n(v):
    nrm = jnp.sqrt(jnp.sum(v * v, axis=1, keepdims=True))
    return v / jnp.maximum(nrm, 1e-12)


def _fill_rows0(rows, val16):
    def fb(i, c):
        rows[0, 0, i, pl.ds(0, 16)] = val16
        rows[0, 0, i, pl.ds(16, 16)] = val16
        return c
    lax.fori_loop(0, CH, fb, 0)


def _zero_acc(rows, acc, row0):
    def zc(i, c):
        pltpu.sync_copy(rows.at[0, 0], acc.at[pl.ds(row0 + i * CH, CH)])
        return c
    lax.fori_loop(0, NZ, zc, 0)


def _seg_pipeline(tbl, cidx, ridx, core, nb0, cbk, rbk, rows, acc,
                  gsems, ssems, isems):
    def _gathers(bank):
        return [pltpu.async_copy(tbl.at[cbk.at[bank, b]],
                                 rows.at[bank, b], gsems[bank])
                for b in range(K)]

    def _drain_gathers(bank):
        for b in range(K):
            pltpu.make_async_copy(tbl.at[cbk.at[bank, b]],
                                  rows.at[bank, b], gsems[bank]).wait()

    def _scatters(bank):
        return [pltpu.async_copy(rows.at[bank, b],
                                 acc.at[rbk.at[bank, b]], ssems[bank],
                                 add=True)
                for b in range(K)]

    for bank in range(2):
        off = nb0 + bank * K
        pltpu.sync_copy(cidx.at[core, pl.ds(off, K)], cbk.at[bank])
        pltpu.sync_copy(ridx.at[core, pl.ds(off, K)], rbk.at[bank])
        _gathers(bank)

    def sup(s, c):
        sds = []
        cst = []
        for bank in range(2):
            _drain_gathers(bank)
            sds.append(_scatters(bank))
            nxt = nb0 + (2 * s + 2 + bank) * K
            cst.append(pltpu.async_copy(cidx.at[core, pl.ds(nxt, K)],
                                        cbk.at[bank], isems[bank]))
        rst = []
        for bank in range(2):
            for d in sds[bank]:
                d.wait()
            nxt = nb0 + (2 * s + 2 + bank) * K
            rst.append(pltpu.async_copy(ridx.at[core, pl.ds(nxt, K)],
                                        rbk.at[bank], isems[bank]))
            cst[bank].wait()
            _gathers(bank)
        for d in rst:
            d.wait()
        return c
    lax.fori_loop(0, NSUP - 1, sup, 0)

    for bank in range(2):
        _drain_gathers(bank)
        for d in _scatters(bank):
            d.wait()


def _cnt_pipeline(src, ridx, core, nb0, rbk, acc, ssems, isems):
    def _scat(bank):
        return [pltpu.async_copy(src, acc.at[rbk.at[bank, b]], ssems[bank],
                                 add=True)
                for b in range(K)]

    for bank in range(2):
        pltpu.sync_copy(ridx.at[core, pl.ds(nb0 + bank * K, K)],
                        rbk.at[bank])

    def sup(s, c):
        sds = [_scat(0), _scat(1)]
        rst = []
        for bank in range(2):
            for d in sds[bank]:
                d.wait()
            nxt = nb0 + (2 * s + 2 + bank) * K
            rst.append(pltpu.async_copy(ridx.at[core, pl.ds(nxt, K)],
                                        rbk.at[bank], isems[bank]))
        for d in rst:
            d.wait()
        return c
    lax.fori_loop(0, NSUP - 1, sup, 0)

    for bank in range(2):
        for d in _scat(bank):
            d.wait()


def _base_seg_body(table2, cidx, ridx, out, cbk, rbk, rows, acc,
                   gsem0, gsem1, ssem0, ssem1, isem0, isem1):
    core = lax.axis_index("c")
    sid = lax.axis_index("s")
    gsems = (gsem0, gsem1)
    ssems = (ssem0, ssem1)
    isems = (isem0, isem1)
    row0 = sid * RPT
    nb0 = sid * NB
    zero16 = jnp.zeros((16,), jnp.float32)

    _fill_rows0(rows, zero16)
    _zero_acc(rows, acc, row0)
    plsc.subcore_barrier()
    _seg_pipeline(table2.at[core], cidx, ridx, core, nb0, cbk, rbk, rows,
                  acc, gsems, ssems, isems)
    plsc.subcore_barrier()
    pltpu.sync_copy(acc.at[pl.ds(row0, RPT)],
                    out.at[core, pl.ds(row0, RPT)])


def _counts_body(ridx, out, rbk, ones, zbuf, acc, ssem0, ssem1, isem0,
                 isem1):
    core = lax.axis_index("c")
    sid = lax.axis_index("s")
    ssems = (ssem0, ssem1)
    isems = (isem0, isem1)
    row0 = sid * RPT
    nb0 = sid * NB
    zero16 = jnp.zeros((16,), jnp.float32)
    one16 = jnp.ones((16,), jnp.float32)

    def fill(i, c):
        zbuf[i, pl.ds(0, 16)] = zero16
        ones[i, pl.ds(0, 16)] = one16
        return c
    lax.fori_loop(0, CH, fill, 0)

    def zc(i, c):
        pltpu.sync_copy(zbuf, acc.at[pl.ds(row0 + i * CH, CH)])
        return c
    lax.fori_loop(0, NZ, zc, 0)
    plsc.subcore_barrier()
    _cnt_pipeline(ones, ridx, core, nb0, rbk, acc, ssems, isems)
    plsc.subcore_barrier()
    pltpu.sync_copy(acc.at[pl.ds(row0, RPT)], out.at[core, pl.ds(row0, RPT)])


def _deep_seg_body(table2, cidx, ridx, out, cbk, rbk, rows, acc,
                   gsem0, gsem1, ssem0, ssem1, isem0, isem1):
    core = lax.axis_index("c")
    sid = lax.axis_index("s")
    gsems = (gsem0, gsem1)
    ssems = (ssem0, ssem1)
    isems = (isem0, isem1)
    row0 = sid * RPT
    nb0 = sid * NB
    zero16 = jnp.zeros((16,), jnp.float32)

    for p in range(2):
        _fill_rows0(rows, zero16)
        _zero_acc(rows, acc, row0)
        plsc.subcore_barrier()
        _seg_pipeline(table2.at[p], cidx, ridx, core, nb0, cbk, rbk, rows,
                      acc, gsems, ssems, isems)
        plsc.subcore_barrier()
        pltpu.sync_copy(acc.at[pl.ds(row0, RPT)],
                        out.at[p, core, pl.ds(row0, RPT)])


def _make_sc_kernels():
    mesh = plsc.VectorSubcoreMesh(core_axis_name="c", subcore_axis_name="s")
    params = pltpu.CompilerParams(use_tc_tiling_on_sc=False)
    scratch = [
        pltpu.VMEM((2, K, CH), jnp.int32),
        pltpu.VMEM((2, K, CH), jnp.int32),
        pltpu.VMEM((2, K, CH, H), jnp.float32),
        pltpu.VMEM_SHARED((ACC_N, H), jnp.float32),
        pltpu.SemaphoreType.DMA,
        pltpu.SemaphoreType.DMA,
        pltpu.SemaphoreType.DMA,
        pltpu.SemaphoreType.DMA,
        pltpu.SemaphoreType.DMA,
        pltpu.SemaphoreType.DMA,
    ]
    base_seg = pl.kernel(
        _base_seg_body,
        out_type=jax.ShapeDtypeStruct((2, ACC_N, H), jnp.float32),
        mesh=mesh,
        compiler_params=params,
        scratch_types=scratch,
    )
    counts = pl.kernel(
        _counts_body,
        out_type=jax.ShapeDtypeStruct((2, ACC_N, 16), jnp.float32),
        mesh=mesh,
        compiler_params=params,
        scratch_types=[
            pltpu.VMEM((2, K, CH), jnp.int32),
            pltpu.VMEM((CH, 16), jnp.float32),
            pltpu.VMEM((CH, 16), jnp.float32),
            pltpu.VMEM_SHARED((ACC_N, 16), jnp.float32),
            pltpu.SemaphoreType.DMA,
            pltpu.SemaphoreType.DMA,
            pltpu.SemaphoreType.DMA,
            pltpu.SemaphoreType.DMA,
        ],
    )
    deep_seg = pl.kernel(
        _deep_seg_body,
        out_type=jax.ShapeDtypeStruct((2, 2, ACC_N, H), jnp.float32),
        mesh=mesh,
        compiler_params=params,
        scratch_types=scratch,
    )
    return base_seg, counts, deep_seg


def _proj_body(x_ref, w_ref, b_ref, zt_ref, xs_ref):
    y = jnp.dot(x_ref[...], w_ref[...], preferred_element_type=jnp.float32)
    y = y + b_ref[...]
    zt_ref[0] = y[:, 0:H]
    zt_ref[1] = y[:, H:2 * H]
    xs_ref[...] = y[:, 2 * H:]


def _base_body(s0_ref, cnt_ref, xs_ref, h_ref, dinv_ref):
    cp0 = cnt_ref[0, :, 0:1]
    cn0 = cnt_ref[1, :, 0:1]
    ip = 1.0 / jnp.maximum(cp0, 1.0)
    inn = 1.0 / jnp.maximum(cn0, 1.0)
    ia = 1.0 / jnp.maximum(cp0 + cn0, 1.0)
    h_ref[0] = _l2n(s0_ref[0] * ip + xs_ref[:, 0:H])
    h_ref[1] = _l2n(s0_ref[1] * inn + xs_ref[:, H:])
    dinv_ref[...] = jnp.concatenate(
        [ip, inn, ia, jnp.zeros((ip.shape[0], H - 3), jnp.float32)], axis=1)


def _deep_math(s4_ref, dinv_ref, h_ref, wp_ref, bp_ref, wn_ref, bn_ref):
    dp = dinv_ref[:, 0:1]
    dn = dinv_ref[:, 1:2]
    da = dinv_ref[:, 2:3]
    sa0, sa1 = s4_ref[0, 0], s4_ref[0, 1]
    sb0, sb1 = s4_ref[1, 0], s4_ref[1, 1]
    mp_p = sa0 * dp
    mn_p = sa1 * dn
    mp_n = sb0 * dp
    mn_n = sb1 * dn
    ma_p = (sa0 + sa1) * da
    ma_n = (sb0 + sb1) * da
    hp, hn = h_ref[0], h_ref[1]
    catp = jnp.concatenate([mp_p, mn_n, mp_n, mn_p, hp, hn, ma_p], axis=1)
    catn = jnp.concatenate([mn_p, mp_n, mn_n, mp_p, hn, hp, ma_n], axis=1)
    hp2 = _l2n(jnp.dot(catp, wp_ref[...], preferred_element_type=jnp.float32)
               + bp_ref[...])
    hn2 = _l2n(jnp.dot(catn, wn_ref[...], preferred_element_type=jnp.float32)
               + bn_ref[...])
    return hp2, hn2


def _deep_mid_body(s4_ref, dinv_ref, h_ref, wp_ref, bp_ref, wn_ref, bn_ref,
                   o_ref):
    hp2, hn2 = _deep_math(s4_ref, dinv_ref, h_ref, wp_ref, bp_ref, wn_ref,
                          bn_ref)
    o_ref[0] = hp2
    o_ref[1] = hn2


def _deep_final_body(s4_ref, dinv_ref, h_ref, wp_ref, bp_ref, wn_ref, bn_ref,
                     o_ref):
    hp2, hn2 = _deep_math(s4_ref, dinv_ref, h_ref, wp_ref, bp_ref, wn_ref,
                          bn_ref)
    o_ref[...] = jnp.concatenate([hp2, hn2], axis=1)


def _spec_rows(shape):
    lead = shape[:-2]
    blk = lead + (BLK, shape[-1])

    def imap(i):
        return (0,) * len(lead) + (i, 0)
    return pl.BlockSpec(blk, imap)


def _spec_full(shape):
    return pl.BlockSpec(shape, lambda i: (0,) * len(shape))


def _proj_call(x, w, b):
    return pl.pallas_call(
        _proj_body,
        grid=(GRID,),
        in_specs=[_spec_rows(x.shape), _spec_full(w.shape), _spec_full(b.shape)],
        out_specs=[_spec_rows((2, N, H)), _spec_rows((N, 2 * H))],
        out_shape=[jax.ShapeDtypeStruct((2, N, H), jnp.float32),
                   jax.ShapeDtypeStruct((N, 2 * H), jnp.float32)],
    )(x, w, b)


def _base_call(s0, cnt, xs):
    return pl.pallas_call(
        _base_body,
        grid=(GRID,),
        in_specs=[_spec_rows(s0.shape), _spec_rows(cnt.shape),
                  _spec_rows(xs.shape)],
        out_specs=[_spec_rows((2, N, H)), _spec_rows((N, H))],
        out_shape=[jax.ShapeDtypeStruct((2, N, H), jnp.float32),
                   jax.ShapeDtypeStruct((N, H), jnp.float32)],
    )(s0, cnt, xs)


def _deep_call(body, out_shape, s4, dinv, h, wp, bp, wn, bn):
    return pl.pallas_call(
        body,
        grid=(GRID,),
        in_specs=[_spec_rows(s4.shape), _spec_rows(dinv.shape),
                  _spec_rows(h.shape),
                  _spec_full(wp.shape), _spec_full(bp.shape),
                  _spec_full(wn.shape), _spec_full(bn.shape)],
        out_specs=_spec_rows(out_shape),
        out_shape=jax.ShapeDtypeStruct(out_shape, jnp.float32),
    )(s4, dinv, h, wp, bp, wn, bn)


def kernel(x, edge_index_pos, edge_index_neg, W_pos_base, b_pos_base,
           W_neg_base, b_neg_base, W_pos_deep_0, b_pos_deep_0, W_pos_deep_1,
           b_pos_deep_1, W_neg_deep_0, b_neg_deep_0, W_neg_deep_1,
           b_neg_deep_1):
    base_seg, counts, deep_seg = _make_sc_kernels()

    eip = edge_index_pos.astype(jnp.int32)
    ein = edge_index_neg.astype(jnp.int32)
    rp, cp = eip[0], eip[1]
    rn, cn = ein[0], ein[1]

    pad = E_PAD - E
    cpad = jnp.zeros((pad,), jnp.int32)
    rpad = jnp.full((pad,), N, jnp.int32)
    cidx = jnp.stack([jnp.concatenate([cp, cpad]),
                      jnp.concatenate([cn, cpad])]).reshape(2, TOT_NB, CH)
    ridx = jnp.stack([jnp.concatenate([rp, rpad]),
                      jnp.concatenate([rn, rpad])]).reshape(2, TOT_NB, CH)

    wcat = jnp.concatenate([W_pos_base[:D], W_neg_base[:D],
                            W_pos_base[D:], W_neg_base[D:]], axis=1)
    bcat = jnp.concatenate([jnp.zeros((2 * H,), jnp.float32),
                            b_pos_base, b_neg_base]).reshape(1, 4 * H)
    zt, xs = _proj_call(x, wcat, bcat)

    cnt = counts(ridx)
    s0 = base_seg(zt, cidx, ridx)
    h, dinv = _base_call(s0, cnt, xs)

    wp = [W_pos_deep_0, W_pos_deep_1]
    bp = [b_pos_deep_0.reshape(1, H), b_pos_deep_1.reshape(1, H)]
    wn = [W_neg_deep_0, W_neg_deep_1]
    bn = [b_neg_deep_0.reshape(1, H), b_neg_deep_1.reshape(1, H)]

    for i in range(2):
        s4 = deep_seg(h, cidx, ridx)
        if i == 0:
            h = _deep_call(_deep_mid_body, (2, N, H), s4, dinv, h,
                           wp[i], bp[i], wn[i], bn[i])
        else:
            out = _deep_call(_deep_final_body, (N, 2 * H), s4, dinv, h,
                             wp[i], bp[i], wn[i], bn[i])
    return out


# speedup vs baseline: 1.3730x; 1.3227x over previous
import jax
import jax.numpy as jnp
from jax import lax
from jax.experimental import pallas as pl
from jax.experimental.pallas import tpu as pltpu
from jax.experimental.pallas import tpu_sc as plsc

N = 50000
D = 128
E = 400000
H = 32

CH = 112
NB = 224
PE = CH * NB
E_PAD = 16 * PE
TOT_NB = 16 * NB
K = 4
NGRP = NB // K
NSUP = NGRP // 2
ACC_N = 50176
RPT = ACC_N // 16
NZ = RPT // CH

N4 = N // 4
ACC4 = ACC_N // 4
BLK = 448
GRID = ACC4 // BLK


def _fill_rows0(rows, val16):
    def fb(i, c):
        rows[0, 0, i, pl.ds(0, 16)] = val16
        rows[0, 0, i, pl.ds(16, 16)] = val16
        return c
    lax.fori_loop(0, CH, fb, 0)


def _zero_acc(rows, acc, row0):
    def zc(i, c):
        pltpu.sync_copy(rows.at[0, 0], acc.at[pl.ds(row0 + i * CH, CH)])
        return c
    lax.fori_loop(0, NZ, zc, 0)


def _seg_pipeline(tbl, cidx, ridx, core, nb0, cbk, rbk, rows, acc,
                  gsems, ssems, isems):
    def _gathers(bank):
        return [pltpu.async_copy(tbl.at[cbk.at[bank, b]],
                                 rows.at[bank, b], gsems[bank])
                for b in range(K)]

    def _drain_gathers(bank):
        for b in range(K):
            pltpu.make_async_copy(tbl.at[cbk.at[bank, b]],
                                  rows.at[bank, b], gsems[bank]).wait()

    def _scatters(bank):
        return [pltpu.async_copy(rows.at[bank, b],
                                 acc.at[rbk.at[bank, b]], ssems[bank],
                                 add=True)
                for b in range(K)]

    for bank in range(2):
        off = nb0 + bank * K
        pltpu.sync_copy(cidx.at[core, pl.ds(off, K)], cbk.at[bank])
        pltpu.sync_copy(ridx.at[core, pl.ds(off, K)], rbk.at[bank])
        _gathers(bank)

    def sup(s, c):
        sds = []
        cst = []
        for bank in range(2):
            _drain_gathers(bank)
            sds.append(_scatters(bank))
            nxt = nb0 + (2 * s + 2 + bank) * K
            cst.append(pltpu.async_copy(cidx.at[core, pl.ds(nxt, K)],
                                        cbk.at[bank], isems[bank]))
        rst = []
        for bank in range(2):
            for d in sds[bank]:
                d.wait()
            nxt = nb0 + (2 * s + 2 + bank) * K
            rst.append(pltpu.async_copy(ridx.at[core, pl.ds(nxt, K)],
                                        rbk.at[bank], isems[bank]))
            cst[bank].wait()
            _gathers(bank)
        for d in rst:
            d.wait()
        return c
    lax.fori_loop(0, NSUP - 1, sup, 0)

    for bank in range(2):
        _drain_gathers(bank)
        for d in _scatters(bank):
            d.wait()


def _cnt_pipeline(src, ridx, core, nb0, rbk, acc, ssems, isems):
    def _scat(bank):
        return [pltpu.async_copy(src, acc.at[rbk.at[bank, b]], ssems[bank],
                                 add=True)
                for b in range(K)]

    for bank in range(2):
        pltpu.sync_copy(ridx.at[core, pl.ds(nb0 + bank * K, K)],
                        rbk.at[bank])

    def sup(s, c):
        sds = [_scat(0), _scat(1)]
        rst = []
        for bank in range(2):
            for d in sds[bank]:
                d.wait()
            nxt = nb0 + (2 * s + 2 + bank) * K
            rst.append(pltpu.async_copy(ridx.at[core, pl.ds(nxt, K)],
                                        rbk.at[bank], isems[bank]))
        for d in rst:
            d.wait()
        return c
    lax.fori_loop(0, NSUP - 1, sup, 0)

    for bank in range(2):
        for d in _scat(bank):
            d.wait()


def _base_seg_body(table2, cidx, ridx, out, cbk, rbk, rows, acc,
                   gsem0, gsem1, ssem0, ssem1, isem0, isem1):
    core = lax.axis_index("c")
    sid = lax.axis_index("s")
    gsems = (gsem0, gsem1)
    ssems = (ssem0, ssem1)
    isems = (isem0, isem1)
    row0 = sid * RPT
    nb0 = sid * NB
    zero16 = jnp.zeros((16,), jnp.float32)

    _fill_rows0(rows, zero16)
    _zero_acc(rows, acc, row0)
    plsc.subcore_barrier()
    _seg_pipeline(table2.at[core], cidx, ridx, core, nb0, cbk, rbk, rows,
                  acc, gsems, ssems, isems)
    plsc.subcore_barrier()
    pltpu.sync_copy(acc.at[pl.ds(row0, RPT)],
                    out.at[core, pl.ds(row0, RPT)])


def _counts_body(ridx, out, rbk, ones, zbuf, acc, ssem0, ssem1, isem0,
                 isem1):
    core = lax.axis_index("c")
    sid = lax.axis_index("s")
    ssems = (ssem0, ssem1)
    isems = (isem0, isem1)
    row0 = sid * RPT
    nb0 = sid * NB
    zero16 = jnp.zeros((16,), jnp.float32)
    one16 = jnp.ones((16,), jnp.float32)

    def fill(i, c):
        zbuf[i, pl.ds(0, 16)] = zero16
        zbuf[i, pl.ds(16, 16)] = zero16
        ones[i, pl.ds(0, 16)] = one16
        ones[i, pl.ds(16, 16)] = one16
        return c
    lax.fori_loop(0, CH, fill, 0)

    def zc(i, c):
        pltpu.sync_copy(zbuf, acc.at[pl.ds(row0 + i * CH, CH)])
        return c
    lax.fori_loop(0, NZ, zc, 0)
    plsc.subcore_barrier()
    _cnt_pipeline(ones, ridx, core, nb0, rbk, acc, ssems, isems)
    plsc.subcore_barrier()
    pltpu.sync_copy(acc.at[pl.ds(row0, RPT)], out.at[core, pl.ds(row0, RPT)])


def _deep_seg_body(table2, cidx, ridx, out, cbk, rbk, rows, acc,
                   gsem0, gsem1, ssem0, ssem1, isem0, isem1):
    core = lax.axis_index("c")
    sid = lax.axis_index("s")
    gsems = (gsem0, gsem1)
    ssems = (ssem0, ssem1)
    isems = (isem0, isem1)
    row0 = sid * RPT
    nb0 = sid * NB
    zero16 = jnp.zeros((16,), jnp.float32)

    for p in range(2):
        _fill_rows0(rows, zero16)
        _zero_acc(rows, acc, row0)
        plsc.subcore_barrier()
        _seg_pipeline(table2.at[p], cidx, ridx, core, nb0, cbk, rbk, rows,
                      acc, gsems, ssems, isems)
        plsc.subcore_barrier()
        pltpu.sync_copy(acc.at[pl.ds(row0, RPT)],
                        out.at[p, core, pl.ds(row0, RPT)])


def _make_sc_kernels():
    mesh = plsc.VectorSubcoreMesh(core_axis_name="c", subcore_axis_name="s")
    params = pltpu.CompilerParams(use_tc_tiling_on_sc=False)
    scratch = [
        pltpu.VMEM((2, K, CH), jnp.int32),
        pltpu.VMEM((2, K, CH), jnp.int32),
        pltpu.VMEM((2, K, CH, H), jnp.float32),
        pltpu.VMEM_SHARED((ACC_N, H), jnp.float32),
        pltpu.SemaphoreType.DMA,
        pltpu.SemaphoreType.DMA,
        pltpu.SemaphoreType.DMA,
        pltpu.SemaphoreType.DMA,
        pltpu.SemaphoreType.DMA,
        pltpu.SemaphoreType.DMA,
    ]
    base_seg = pl.kernel(
        _base_seg_body,
        out_type=jax.ShapeDtypeStruct((2, ACC_N, H), jnp.float32),
        mesh=mesh,
        compiler_params=params,
        scratch_types=scratch,
    )
    counts = pl.kernel(
        _counts_body,
        out_type=jax.ShapeDtypeStruct((2, ACC_N, H), jnp.float32),
        mesh=mesh,
        compiler_params=params,
        scratch_types=[
            pltpu.VMEM((2, K, CH), jnp.int32),
            pltpu.VMEM((CH, H), jnp.float32),
            pltpu.VMEM((CH, H), jnp.float32),
            pltpu.VMEM_SHARED((ACC_N, H), jnp.float32),
            pltpu.SemaphoreType.DMA,
            pltpu.SemaphoreType.DMA,
            pltpu.SemaphoreType.DMA,
            pltpu.SemaphoreType.DMA,
        ],
    )
    deep_seg = pl.kernel(
        _deep_seg_body,
        out_type=jax.ShapeDtypeStruct((2, 2, ACC_N, H), jnp.float32),
        mesh=mesh,
        compiler_params=params,
        scratch_types=scratch,
    )
    return base_seg, counts, deep_seg


def _l2n4(v):
    outs = []
    for j in range(4):
        g = v[:, H * j:H * (j + 1)]
        nrm = jnp.sqrt(jnp.sum(g * g, axis=1, keepdims=True))
        outs.append(g / jnp.maximum(nrm, 1e-12))
    return jnp.concatenate(outs, axis=1)


def _proj_body(x4_ref, w4_ref, bxp_ref, bxn_ref, zt_ref, xs_ref):
    xb = x4_ref[...]
    zt_ref[0] = jnp.dot(xb, w4_ref[0], preferred_element_type=jnp.float32)
    zt_ref[1] = jnp.dot(xb, w4_ref[1], preferred_element_type=jnp.float32)
    xs_ref[0] = jnp.dot(xb, w4_ref[2],
                        preferred_element_type=jnp.float32) + bxp_ref[...]
    xs_ref[1] = jnp.dot(xb, w4_ref[3],
                        preferred_element_type=jnp.float32) + bxn_ref[...]


def _base_body(s0_ref, cnt_ref, xs_ref, h_ref):
    ip = 1.0 / jnp.maximum(cnt_ref[0], 1.0)
    inn = 1.0 / jnp.maximum(cnt_ref[1], 1.0)
    h_ref[0] = _l2n4(s0_ref[0] * ip + xs_ref[0])
    h_ref[1] = _l2n4(s0_ref[1] * inn + xs_ref[1])


def _deep_body(s4_ref, cnt_ref, h_ref, wp_ref, bp_ref, wn_ref, bn_ref,
               o_ref):
    c0 = cnt_ref[0]
    c1 = cnt_ref[1]
    ip = 1.0 / jnp.maximum(c0, 1.0)
    inn = 1.0 / jnp.maximum(c1, 1.0)
    ia = 1.0 / jnp.maximum(c0 + c1, 1.0)
    sa0, sa1 = s4_ref[0, 0], s4_ref[0, 1]
    sb0, sb1 = s4_ref[1, 0], s4_ref[1, 1]
    mp_p = sa0 * ip
    mn_p = sa1 * inn
    mp_n = sb0 * ip
    mn_n = sb1 * inn
    ma_p = (sa0 + sa1) * ia
    ma_n = (sb0 + sb1) * ia
    hp, hn = h_ref[0], h_ref[1]
    partsp = [mp_p, mn_n, mp_n, mn_p, hp, hn, ma_p]
    partsn = [mn_p, mp_n, mn_n, mp_p, hn, hp, ma_n]
    yp = bp_ref[...]
    yn = bn_ref[...]
    for g in range(7):
        yp = yp + jnp.dot(partsp[g], wp_ref[4 * H * g:4 * H * (g + 1)],
                          preferred_element_type=jnp.float32)
        yn = yn + jnp.dot(partsn[g], wn_ref[4 * H * g:4 * H * (g + 1)],
                          preferred_element_type=jnp.float32)
    o_ref[0] = _l2n4(yp)
    o_ref[1] = _l2n4(yn)


def _spec_rows(shape):
    lead = shape[:-2]
    blk = lead + (BLK, shape[-1])

    def imap(i):
        return (0,) * len(lead) + (i, 0)
    return pl.BlockSpec(blk, imap)


def _spec_full(shape):
    return pl.BlockSpec(shape, lambda i: (0,) * len(shape))


def _proj_call(x4, w4, bxp, bxn):
    return pl.pallas_call(
        _proj_body,
        grid=(GRID,),
        in_specs=[_spec_rows(x4.shape), _spec_full(w4.shape),
                  _spec_full(bxp.shape), _spec_full(bxn.shape)],
        out_specs=[_spec_rows((2, ACC4, 4 * H)), _spec_rows((2, ACC4, 4 * H))],
        out_shape=[jax.ShapeDtypeStruct((2, ACC4, 4 * H), jnp.float32),
                   jax.ShapeDtypeStruct((2, ACC4, 4 * H), jnp.float32)],
    )(x4, w4, bxp, bxn)


def _base_call(s0v, cntv, xs):
    return pl.pallas_call(
        _base_body,
        grid=(GRID,),
        in_specs=[_spec_rows(s0v.shape), _spec_rows(cntv.shape),
                  _spec_rows(xs.shape)],
        out_specs=_spec_rows((2, ACC4, 4 * H)),
        out_shape=jax.ShapeDtypeStruct((2, ACC4, 4 * H), jnp.float32),
    )(s0v, cntv, xs)


def _deep_call(s4v, cntv, h4, wp, bp, wn, bn):
    return pl.pallas_call(
        _deep_body,
        grid=(GRID,),
        in_specs=[_spec_rows(s4v.shape), _spec_rows(cntv.shape),
                  _spec_rows(h4.shape),
                  _spec_full(wp.shape), _spec_full(bp.shape),
                  _spec_full(wn.shape), _spec_full(bn.shape)],
        out_specs=_spec_rows((2, ACC4, 4 * H)),
        out_shape=jax.ShapeDtypeStruct((2, ACC4, 4 * H), jnp.float32),
    )(s4v, cntv, h4, wp, bp, wn, bn)


def kernel(x, edge_index_pos, edge_index_neg, W_pos_base, b_pos_base,
           W_neg_base, b_neg_base, W_pos_deep_0, b_pos_deep_0, W_pos_deep_1,
           b_pos_deep_1, W_neg_deep_0, b_neg_deep_0, W_neg_deep_1,
           b_neg_deep_1):
    base_seg, counts, deep_seg = _make_sc_kernels()

    eip = edge_index_pos.astype(jnp.int32)
    ein = edge_index_neg.astype(jnp.int32)
    rp, cp = eip[0], eip[1]
    rn, cn = ein[0], ein[1]

    pad = E_PAD - E
    cpad = jnp.zeros((pad,), jnp.int32)
    rpad = jnp.full((pad,), N, jnp.int32)
    cidx = jnp.stack([jnp.concatenate([cp, cpad]),
                      jnp.concatenate([cn, cpad])]).reshape(2, TOT_NB, CH)
    ridx = jnp.stack([jnp.concatenate([rp, rpad]),
                      jnp.concatenate([rn, rpad])]).reshape(2, TOT_NB, CH)

    eye4 = jnp.eye(4, dtype=jnp.float32)

    def bd4(w):
        return jnp.kron(eye4, w)

    x4 = jnp.pad(x.reshape(N4, 4 * D), ((0, ACC4 - N4), (0, 0)))
    w4 = jnp.stack([bd4(W_pos_base[:D]), bd4(W_neg_base[:D]),
                    bd4(W_pos_base[D:]), bd4(W_neg_base[D:])])
    bxp = jnp.tile(b_pos_base, 4).reshape(1, 4 * H)
    bxn = jnp.tile(b_neg_base, 4).reshape(1, 4 * H)
    zt4, xs4 = _proj_call(x4, w4, bxp, bxn)

    cnt = counts(ridx)
    cntv = cnt.reshape(2, ACC4, 4 * H)
    s0 = base_seg(zt4.reshape(2, ACC_N, H), cidx, ridx)
    h4 = _base_call(s0.reshape(2, ACC4, 4 * H), cntv, xs4)

    def bd7(w):
        return jnp.concatenate([bd4(w[H * g:H * (g + 1)])
                                for g in range(7)], axis=0)

    wp = [bd7(W_pos_deep_0), bd7(W_pos_deep_1)]
    bp = [jnp.tile(b_pos_deep_0, 4).reshape(1, 4 * H),
          jnp.tile(b_pos_deep_1, 4).reshape(1, 4 * H)]
    wn = [bd7(W_neg_deep_0), bd7(W_neg_deep_1)]
    bn = [jnp.tile(b_neg_deep_0, 4).reshape(1, 4 * H),
          jnp.tile(b_neg_deep_1, 4).reshape(1, 4 * H)]

    for i in range(2):
        s4 = deep_seg(h4.reshape(2, ACC_N, H), cidx, ridx)
        h4 = _deep_call(s4.reshape(2, 2, ACC4, 4 * H), cntv, h4,
                        wp[i], bp[i], wn[i], bn[i])
    return jnp.concatenate([h4[0].reshape(ACC_N, H)[:N],
                            h4[1].reshape(ACC_N, H)[:N]], axis=1)
